# baseline (speedup 1.0000x reference)
import jax
import jax.numpy as jnp
from jax.experimental import pallas as pl
from jax.experimental.pallas import tpu as pltpu

_CH = 256
_NBUF = 10


def _body(x_hbm, o_ref, buf, best, sems, *, nsteps):
    i = pl.program_id(0)

    def copy(slot, step):
        return pltpu.make_async_copy(
            x_hbm.at[pl.ds(step * _CH, _CH), :],
            buf.at[slot],
            sems.at[slot],
        )

    @pl.when(i == 0)
    def _():
        best[0] = -jnp.inf
        for k in range(_NBUF):
            copy(k, k).start()

    slot = jax.lax.rem(i, _NBUF)
    copy(slot, i).wait()
    xb = buf[slot]
    sq = jnp.sum(xb * xb, axis=1, keepdims=True)
    bv = jnp.max(sq)

    @pl.when(i + _NBUF < nsteps)
    def _():
        copy(slot, i + _NBUF).start()

    @pl.when(bv > best[0])
    def _():
        best[0] = bv
        row_ids = jax.lax.broadcasted_iota(jnp.int32, (_CH, 1), 0)
        bi = jnp.min(jnp.where(sq == bv, row_ids, _CH))
        onehot = (row_ids == bi).astype(xb.dtype)
        o_ref[...] = jnp.sum(xb * onehot, axis=0, keepdims=True)


def kernel(x):
    rows, cols = x.shape
    nsteps = rows // _CH
    import functools
    return pl.pallas_call(
        functools.partial(_body, nsteps=nsteps),
        grid=(nsteps,),
        in_specs=[pl.BlockSpec(memory_space=pl.ANY)],
        out_specs=pl.BlockSpec((1, cols), lambda i: (0, 0)),
        out_shape=jax.ShapeDtypeStruct((1, cols), x.dtype),
        scratch_shapes=[
            pltpu.VMEM((_NBUF, _CH, cols), jnp.float32),
            pltpu.SMEM((1,), jnp.float32),
            pltpu.SemaphoreType.DMA((_NBUF,)),
        ],
        compiler_params=pltpu.CompilerParams(
            dimension_semantics=("arbitrary",),
        ),
    )(x)


# speedup vs baseline: 1.0511x; 1.0511x over previous
import jax
import jax.numpy as jnp
from jax.experimental import pallas as pl
from jax.experimental.pallas import tpu as pltpu

_CH = 256
_NBUF = 8


def _body(x_hbm, o_ref, buf, best, sems, *, nsteps):
    i = pl.program_id(0)

    def copy(slot, step):
        return pltpu.make_async_copy(
            x_hbm.at[pl.ds(step * _CH, _CH), :],
            buf.at[slot],
            sems.at[slot],
        )

    @pl.when(i == 0)
    def _():
        best[0] = -jnp.inf
        for k in range(_NBUF):
            copy(k, k).start()

    slot = jax.lax.rem(i, _NBUF)
    copy(slot, i).wait()
    xb = buf[slot]
    sq = jnp.sum(xb * xb, axis=1, keepdims=True)
    bv = jnp.max(sq)

    @pl.when(i + _NBUF < nsteps)
    def _():
        copy(slot, i + _NBUF).start()

    @pl.when(bv > best[0])
    def _():
        best[0] = bv
        row_ids = jax.lax.broadcasted_iota(jnp.int32, (_CH, 1), 0)
        bi = jnp.min(jnp.where(sq == bv, row_ids, _CH))
        onehot = (row_ids == bi).astype(xb.dtype)
        o_ref[...] = jnp.sum(xb * onehot, axis=0, keepdims=True)


def kernel(x):
    rows, cols = x.shape
    nsteps = rows // _CH
    import functools
    return pl.pallas_call(
        functools.partial(_body, nsteps=nsteps),
        grid=(nsteps,),
        in_specs=[pl.BlockSpec(memory_space=pl.ANY)],
        out_specs=pl.BlockSpec((1, cols), lambda i: (0, 0)),
        out_shape=jax.ShapeDtypeStruct((1, cols), x.dtype),
        scratch_shapes=[
            pltpu.VMEM((_NBUF, _CH, cols), jnp.float32),
            pltpu.SMEM((1,), jnp.float32),
            pltpu.SemaphoreType.DMA((_NBUF,)),
        ],
        compiler_params=pltpu.CompilerParams(
            dimension_semantics=("arbitrary",),
        ),
    )(x)
